# baseline (speedup 1.0000x reference)
import functools

import jax
import jax.numpy as jnp
from jax import lax
from jax.experimental import pallas as pl
from jax.experimental.pallas import tpu as pltpu
from jax.experimental.pallas import tpu_sc as plsc

_SC_NC = 2
_SC_NS = 16
_SC_NW = _SC_NC * _SC_NS


def _gather_body(table_hbm, hidx_hbm, tidx_hbm, outh_hbm, outt_hbm,
                 hidx_v, hrows_v, tidx_v, trows_v, hsem, tsem):
    wid = lax.axis_index("s") * _SC_NC + lax.axis_index("c")
    bpw = hidx_v.shape[0]
    base = wid * bpw
    pltpu.sync_copy(hidx_hbm.at[pl.ds(base, bpw)], hidx_v)
    pltpu.sync_copy(tidx_hbm.at[pl.ds(base, bpw)], tidx_v)
    ch = pltpu.async_copy(table_hbm.at[hidx_v], hrows_v, hsem)
    ct = pltpu.async_copy(table_hbm.at[tidx_v], trows_v, tsem)
    ch.wait()
    ct.wait()
    pltpu.sync_copy(hrows_v, outh_hbm.at[pl.ds(base, bpw)])
    pltpu.sync_copy(trows_v, outt_hbm.at[pl.ds(base, bpw)])


def _sc_gather(table, hidx, tidx):
    B = hidx.shape[0]
    D = table.shape[1]
    bpw = B // _SC_NW
    run = pl.kernel(
        _gather_body,
        out_type=[
            jax.ShapeDtypeStruct((B, D), jnp.float32),
            jax.ShapeDtypeStruct((B, D), jnp.float32),
        ],
        mesh=plsc.VectorSubcoreMesh(
            core_axis_name="c", subcore_axis_name="s",
            num_cores=_SC_NC, num_subcores=_SC_NS,
        ),
        scratch_types=[
            pltpu.VMEM((bpw,), jnp.int32),
            pltpu.VMEM((bpw, D), jnp.float32),
            pltpu.VMEM((bpw,), jnp.int32),
            pltpu.VMEM((bpw, D), jnp.float32),
            pltpu.SemaphoreType.DMA,
            pltpu.SemaphoreType.DMA,
        ],
        compiler_params=pltpu.CompilerParams(use_tc_tiling_on_sc=False),
    )
    return run(table, hidx, tidx)


def _score_body(ent_ref, rule_ref, qr_ref, depth_ref, h_ref, t_ref, out_ref):
    E = ent_ref[...]
    R = rule_ref[...]
    N = E.shape[0]
    NR = R.shape[0]
    H = h_ref[...]
    T = t_ref[...]

    dn = (((1,), (1,)), ((), ()))
    GH = lax.dot_general(H, E, dn, preferred_element_type=jnp.float32)
    GT = lax.dot_general(T, E, dn, preferred_element_type=jnp.float32)
    nH = jnp.sum(H * H, axis=1, keepdims=True)
    nT = jnp.sum(T * T, axis=1, keepdims=True)
    ones = jnp.ones((1, E.shape[1]), jnp.float32)
    nE = lax.dot_general(ones, E * E, dn, preferred_element_type=jnp.float32)
    PE = lax.dot_general(R, E, dn, preferred_element_type=jnp.float32)
    PH = lax.dot_general(H, R, dn, preferred_element_type=jnp.float32)

    qr = qr_ref[0]
    rq = rule_ref[pl.ds(qr, 1), :]
    nq = jnp.sum(rq * rq)
    pEq = lax.dot_general(rq, E, dn, preferred_element_type=jnp.float32)
    pTq = lax.dot_general(T, rq, dn, preferred_element_type=jnp.float32)

    base = nH + nE - 2.0 * GH
    s1 = jnp.zeros_like(base)
    for rel in range(NR):
        nr = jnp.sum(R[rel : rel + 1, :] ** 2)
        d2 = base + (2.0 * PH[:, rel : rel + 1] + nr) - 2.0 * PE[rel : rel + 1, :]
        s1 = s1 - jnp.sqrt(jnp.maximum(d2, 0.0))

    d2q = (nE + 2.0 * pEq) + (nq + nT - 2.0 * pTq) - 2.0 * GT
    s2 = -jnp.sqrt(jnp.maximum(d2q, 0.0))

    score = jnp.sum(s1 * s2, axis=1, keepdims=True) * (1.0 / N)

    dv = H + rq - T
    base_out = -jnp.sqrt(jnp.sum(dv * dv, axis=1, keepdims=True))

    d = depth_ref[0]
    out_ref[...] = jnp.where(d == 0, base_out, score)


def kernel(ent_emb, rule_emb, query_relation, head, tail, depth):
    B = head.shape[0]
    H, T = _sc_gather(
        ent_emb, head.astype(jnp.int32), tail.astype(jnp.int32)
    )
    out = pl.pallas_call(
        _score_body,
        out_shape=jax.ShapeDtypeStruct((B, 1), jnp.float32),
        in_specs=[
            pl.BlockSpec(memory_space=pltpu.VMEM),
            pl.BlockSpec(memory_space=pltpu.VMEM),
            pl.BlockSpec(memory_space=pltpu.SMEM),
            pl.BlockSpec(memory_space=pltpu.SMEM),
            pl.BlockSpec(memory_space=pltpu.VMEM),
            pl.BlockSpec(memory_space=pltpu.VMEM),
        ],
        out_specs=pl.BlockSpec(memory_space=pltpu.VMEM),
    )(
        ent_emb,
        rule_emb,
        query_relation.astype(jnp.int32),
        jnp.asarray(depth, jnp.int32).reshape(1),
        H,
        T,
    )
    return out.reshape(B)


# speedup vs baseline: 1.1763x; 1.1763x over previous
import functools

import jax
import jax.numpy as jnp
from jax import lax
from jax.experimental import pallas as pl
from jax.experimental.pallas import tpu as pltpu
from jax.experimental.pallas import tpu_sc as plsc

_SC_NC = 2
_SC_NS = 16
_SC_NW = _SC_NC * _SC_NS


def _gather_body(table_hbm, hidx_hbm, tidx_hbm, outh_hbm, outt_hbm,
                 idx_v, rows_v, sem):
    wid = lax.axis_index("s") * _SC_NC + lax.axis_index("c")
    bpw = idx_v.shape[0]
    half = _SC_NW // 2

    @pl.when(wid < half)
    def _():
        base = wid * bpw
        pltpu.sync_copy(hidx_hbm.at[pl.ds(base, bpw)], idx_v)
        pltpu.async_copy(table_hbm.at[idx_v], rows_v, sem).wait()
        pltpu.sync_copy(rows_v, outh_hbm.at[pl.ds(base, bpw)])

    @pl.when(wid >= half)
    def _():
        base = (wid - half) * bpw
        pltpu.sync_copy(tidx_hbm.at[pl.ds(base, bpw)], idx_v)
        pltpu.async_copy(table_hbm.at[idx_v], rows_v, sem).wait()
        pltpu.sync_copy(rows_v, outt_hbm.at[pl.ds(base, bpw)])


def _sc_gather(table, hidx, tidx):
    B = hidx.shape[0]
    D = table.shape[1]
    bpw = (2 * B) // _SC_NW
    run = pl.kernel(
        _gather_body,
        out_type=[
            jax.ShapeDtypeStruct((B, D), jnp.float32),
            jax.ShapeDtypeStruct((B, D), jnp.float32),
        ],
        mesh=plsc.VectorSubcoreMesh(
            core_axis_name="c", subcore_axis_name="s",
            num_cores=_SC_NC, num_subcores=_SC_NS,
        ),
        scratch_types=[
            pltpu.VMEM((bpw,), jnp.int32),
            pltpu.VMEM((bpw, D), jnp.float32),
            pltpu.SemaphoreType.DMA,
        ],
    )
    return run(table, hidx, tidx)


def _score_body(ent_ref, rule_ref, qr_ref, depth_ref, h_ref, t_ref, out_ref):
    E = ent_ref[...]
    R = rule_ref[...]
    N = E.shape[0]
    NR = R.shape[0]
    D = E.shape[1]
    H = h_ref[:, :D]
    T = t_ref[:, :D]

    dn = (((1,), (1,)), ((), ()))
    GH = lax.dot_general(H, E, dn, preferred_element_type=jnp.float32)
    GT = lax.dot_general(T, E, dn, preferred_element_type=jnp.float32)
    nH = jnp.sum(H * H, axis=1, keepdims=True)
    nT = jnp.sum(T * T, axis=1, keepdims=True)
    ones = jnp.ones((1, E.shape[1]), jnp.float32)
    nE = lax.dot_general(ones, E * E, dn, preferred_element_type=jnp.float32)
    PE = lax.dot_general(R, E, dn, preferred_element_type=jnp.float32)
    PH = lax.dot_general(H, R, dn, preferred_element_type=jnp.float32)

    qr = qr_ref[0]
    rq = rule_ref[pl.ds(qr, 1), :]
    nq = jnp.sum(rq * rq)
    pEq = lax.dot_general(rq, E, dn, preferred_element_type=jnp.float32)
    pTq = lax.dot_general(T, rq, dn, preferred_element_type=jnp.float32)

    base = nH + nE - 2.0 * GH
    d2q = (nE + 2.0 * pEq) + (nq + nT - 2.0 * pTq) - 2.0 * GT
    acc = jnp.zeros_like(base)
    for rel in range(NR):
        nr = jnp.sum(R[rel : rel + 1, :] ** 2)
        d2 = base + (2.0 * PH[:, rel : rel + 1] + nr) - 2.0 * PE[rel : rel + 1, :]
        p = jnp.maximum(d2 * d2q, 1e-35)
        acc = acc + p * lax.rsqrt(p)

    score = jnp.sum(acc, axis=1, keepdims=True) * (1.0 / N)

    dv = H + rq - T
    base_out = -jnp.sqrt(jnp.sum(dv * dv, axis=1, keepdims=True))

    d = depth_ref[0]
    out_ref[...] = jnp.where(d == 0, base_out, score)


def kernel(ent_emb, rule_emb, query_relation, head, tail, depth):
    B = head.shape[0]
    N, D = ent_emb.shape
    ent_pad = jnp.pad(ent_emb, ((0, 0), (0, 128 - D)))
    H, T = _sc_gather(
        ent_pad, head.astype(jnp.int32), tail.astype(jnp.int32)
    )
    out = pl.pallas_call(
        _score_body,
        out_shape=jax.ShapeDtypeStruct((B, 1), jnp.float32),
        in_specs=[
            pl.BlockSpec(memory_space=pltpu.VMEM),
            pl.BlockSpec(memory_space=pltpu.VMEM),
            pl.BlockSpec(memory_space=pltpu.SMEM),
            pl.BlockSpec(memory_space=pltpu.SMEM),
            pl.BlockSpec(memory_space=pltpu.VMEM),
            pl.BlockSpec(memory_space=pltpu.VMEM),
        ],
        out_specs=pl.BlockSpec(memory_space=pltpu.VMEM),
    )(
        ent_emb,
        rule_emb,
        query_relation.astype(jnp.int32),
        jnp.asarray(depth, jnp.int32).reshape(1),
        H,
        T,
    )
    return out.reshape(B)


# speedup vs baseline: 1.3248x; 1.1262x over previous
import functools

import jax
import jax.numpy as jnp
from jax import lax
from jax.experimental import pallas as pl
from jax.experimental.pallas import tpu as pltpu
from jax.experimental.pallas import tpu_sc as plsc

_SC_NC = 2
_SC_NS = 16
_SC_NW = _SC_NC * _SC_NS


def _gather_body(table_hbm, hidx_hbm, tidx_hbm, outh_hbm, outt_hbm,
                 idx_v, rows_v, sem):
    wid = lax.axis_index("s") * _SC_NC + lax.axis_index("c")
    bpw = idx_v.shape[0]
    half = _SC_NW // 2

    @pl.when(wid < half)
    def _():
        base = wid * bpw
        pltpu.sync_copy(hidx_hbm.at[pl.ds(base, bpw)], idx_v)
        pltpu.async_copy(table_hbm.at[idx_v], rows_v, sem).wait()
        pltpu.sync_copy(rows_v, outh_hbm.at[pl.ds(base, bpw)])

    @pl.when(wid >= half)
    def _():
        base = (wid - half) * bpw
        pltpu.sync_copy(tidx_hbm.at[pl.ds(base, bpw)], idx_v)
        pltpu.async_copy(table_hbm.at[idx_v], rows_v, sem).wait()
        pltpu.sync_copy(rows_v, outt_hbm.at[pl.ds(base, bpw)])


def _sc_gather(table, hidx, tidx):
    B = hidx.shape[0]
    D = table.shape[1]
    bpw = (2 * B) // _SC_NW
    run = pl.kernel(
        _gather_body,
        out_type=[
            jax.ShapeDtypeStruct((B, D), jnp.float32),
            jax.ShapeDtypeStruct((B, D), jnp.float32),
        ],
        mesh=plsc.VectorSubcoreMesh(
            core_axis_name="c", subcore_axis_name="s",
            num_cores=_SC_NC, num_subcores=_SC_NS,
        ),
        scratch_types=[
            pltpu.VMEM((bpw,), jnp.int32),
            pltpu.VMEM((bpw, D), jnp.float32),
            pltpu.SemaphoreType.DMA,
        ],
    )
    return run(table, hidx, tidx)


def _score_body(ent_ref, rule_ref, qr_ref, depth_ref, h_ref, t_ref, out_ref):
    E = ent_ref[...]
    R = rule_ref[...]
    N = E.shape[0]
    NR = R.shape[0]
    D = E.shape[1]
    H = h_ref[:, :D]
    T = t_ref[:, :D]

    dn = (((1,), (1,)), ((), ()))
    Hm2 = -2.0 * H
    Tm2 = -2.0 * T
    GH2 = lax.dot_general(Hm2, E, dn, preferred_element_type=jnp.float32)
    GT2 = lax.dot_general(Tm2, E, dn, preferred_element_type=jnp.float32)
    ones = jnp.ones((1, D), jnp.float32)
    nH_c = jnp.sum(H * H, axis=1, keepdims=True)
    nT_c = jnp.sum(T * T, axis=1, keepdims=True)
    nE_r = lax.dot_general(ones, E * E, dn, preferred_element_type=jnp.float32)
    PH2 = lax.dot_general(H, 2.0 * R, dn, preferred_element_type=jnp.float32)
    PE2 = lax.dot_general(-2.0 * R, E, dn, preferred_element_type=jnp.float32)
    nr_r = lax.dot_general(ones, R * R, dn, preferred_element_type=jnp.float32)
    PH2nr = PH2 + nr_r

    qr = qr_ref[0]
    rq = rule_ref[pl.ds(qr, 1), :]
    nq = jnp.sum(rq * rq)
    pEq2_r = lax.dot_general(2.0 * rq, E, dn, preferred_element_type=jnp.float32)
    pTq2_c = lax.dot_general(Tm2, rq, dn, preferred_element_type=jnp.float32)

    base = GH2 + nH_c + nE_r
    d2q = GT2 + (nE_r + pEq2_r) + (nT_c + pTq2_c + nq)
    acc = jnp.zeros_like(base)
    for rel in range(NR):
        d2 = (base + PH2nr[:, rel : rel + 1]) + PE2[rel : rel + 1, :]
        p = jnp.maximum(d2 * d2q, 1e-35)
        acc = acc + p * lax.rsqrt(p)

    inv_n = jnp.full((1, N), 1.0 / N, jnp.float32)
    score = lax.dot_general(inv_n, acc, dn, preferred_element_type=jnp.float32)

    dv = H + rq - T
    base_out = -jnp.sqrt(
        lax.dot_general(ones, dv * dv, dn, preferred_element_type=jnp.float32)
    )

    d = depth_ref[0]
    out_ref[...] = jnp.where(d == 0, base_out, score)


def kernel(ent_emb, rule_emb, query_relation, head, tail, depth):
    B = head.shape[0]
    N, D = ent_emb.shape
    ent_pad = jnp.pad(ent_emb, ((0, 0), (0, 128 - D)))
    H, T = _sc_gather(
        ent_pad, head.astype(jnp.int32), tail.astype(jnp.int32)
    )
    out = pl.pallas_call(
        _score_body,
        out_shape=jax.ShapeDtypeStruct((1, B), jnp.float32),
        in_specs=[
            pl.BlockSpec(memory_space=pltpu.VMEM),
            pl.BlockSpec(memory_space=pltpu.VMEM),
            pl.BlockSpec(memory_space=pltpu.SMEM),
            pl.BlockSpec(memory_space=pltpu.SMEM),
            pl.BlockSpec(memory_space=pltpu.VMEM),
            pl.BlockSpec(memory_space=pltpu.VMEM),
        ],
        out_specs=pl.BlockSpec(memory_space=pltpu.VMEM),
    )(
        ent_emb,
        rule_emb,
        query_relation.astype(jnp.int32),
        jnp.asarray(depth, jnp.int32).reshape(1),
        H,
        T,
    )
    return out.reshape(B)
